# Initial kernel scaffold; baseline (speedup 1.0000x reference)
import jax
import jax.numpy as jnp
from jax.experimental import pallas as pl


def kernel(x, edge_index, edge_attr, batch, node_W, node_b, edge_W, edge_b, W1, b1, W2, b2, bn_g, bn_b, gate_W, gate_b, pool_W, pool_b):
    raise NotImplementedError("write your pallas kernel here")


# speedup vs baseline: 3.2831x; 3.2831x over previous
import functools

import jax
import jax.numpy as jnp
from jax import lax
from jax.experimental import pallas as pl
from jax.experimental.pallas import tpu as pltpu
from jax.experimental.pallas import tpu_sc as plsc

_N = 10000
_E = 320000
_H = 128
_G = 64
_L = 4

_NC = 2
_NS = 16
_EPW = _E // (_NC * _NS)
_C = 80
_RPT0 = 640
_RPT1 = _N - (_NS - 1) * _RPT0


def _proj_body(x_ref, w_ref, b_ref, o_ref):
    x = x_ref[...]
    x = jnp.where(jnp.isnan(x), 0.0, x)
    y = jnp.dot(x, w_ref[...], preferred_element_type=jnp.float32)
    o_ref[...] = jnp.maximum(y + b_ref[...], 0.0)


def _proj(x, w, b, blk):
    m, k = x.shape
    h = w.shape[1]
    return pl.pallas_call(
        _proj_body,
        grid=(m // blk,),
        in_specs=[
            pl.BlockSpec((blk, k), lambda i: (i, 0)),
            pl.BlockSpec((k, h), lambda i: (0, 0)),
            pl.BlockSpec((1, h), lambda i: (0, 0)),
        ],
        out_specs=pl.BlockSpec((blk, h), lambda i: (i, 0)),
        out_shape=jax.ShapeDtypeStruct((m, h), jnp.float32),
    )(x, w, b.reshape(1, h))


def _sc_edge_body(h_hbm, ea_hbm, src_hbm, dst_hbm, out_a, out_b,
                  aggr_sh, eav, gv, srcv, dstv, sem_e, sem_g):
    c = lax.axis_index("c")
    s = lax.axis_index("s")

    def _zrow(r, carry):
        for j in range(_H // 16):
            gv[r, pl.ds(j * 16, 16)] = jnp.zeros((16,), jnp.float32)
        return carry

    lax.fori_loop(0, _C, _zrow, 0)
    r0 = pl.multiple_of(s * _RPT0, 8)

    @pl.when(s < _NS - 1)
    def _():
        for k in range(_RPT0 // _C):
            pltpu.sync_copy(gv, aggr_sh.at[pl.ds(r0 + k * _C, _C)])

    @pl.when(s == _NS - 1)
    def _():
        for k in range(_RPT1 // _C):
            pltpu.sync_copy(gv, aggr_sh.at[pl.ds(r0 + k * _C, _C)])

    plsc.subcore_barrier()

    base0 = (c * _NS + s) * _EPW

    def _chunk(i, carry):
        b = pl.multiple_of(base0 + i * _C, 8)
        pltpu.sync_copy(src_hbm.at[pl.ds(b, _C)], srcv)
        pltpu.sync_copy(dst_hbm.at[pl.ds(b, _C)], dstv)
        cp_e = pltpu.async_copy(ea_hbm.at[pl.ds(b, _C)], eav, sem_e)
        cp_g = pltpu.async_copy(h_hbm.at[srcv], gv, sem_g)
        cp_e.wait()
        cp_g.wait()

        def _row(r, c2):
            for j in range(_H // 16):
                sl = pl.ds(j * 16, 16)
                eav[r, sl] = jnp.maximum(eav[r, sl] + gv[r, sl], 0.0)
            return c2

        lax.fori_loop(0, _C, _row, 0)
        pltpu.sync_copy(eav, aggr_sh.at[dstv], add=True)
        return carry

    lax.fori_loop(0, _EPW // _C, _chunk, 0)
    plsc.subcore_barrier()

    @pl.when((c == 0) & (s < _NS - 1))
    def _():
        pltpu.sync_copy(aggr_sh.at[pl.ds(r0, _RPT0)], out_a.at[pl.ds(r0, _RPT0)])

    @pl.when((c == 0) & (s == _NS - 1))
    def _():
        pltpu.sync_copy(aggr_sh.at[pl.ds(r0, _RPT1)], out_a.at[pl.ds(r0, _RPT1)])

    @pl.when((c == 1) & (s < _NS - 1))
    def _():
        pltpu.sync_copy(aggr_sh.at[pl.ds(r0, _RPT0)], out_b.at[pl.ds(r0, _RPT0)])

    @pl.when((c == 1) & (s == _NS - 1))
    def _():
        pltpu.sync_copy(aggr_sh.at[pl.ds(r0, _RPT1)], out_b.at[pl.ds(r0, _RPT1)])


@functools.cache
def _edge_pass_kernel():
    return pl.kernel(
        _sc_edge_body,
        out_type=(jax.ShapeDtypeStruct((_N, _H), jnp.float32),
                  jax.ShapeDtypeStruct((_N, _H), jnp.float32)),
        mesh=plsc.VectorSubcoreMesh(core_axis_name="c", subcore_axis_name="s",
                                    num_cores=_NC, num_subcores=_NS),
        scratch_types=[
            pltpu.VMEM_SHARED((_N, _H), jnp.float32),
            pltpu.VMEM((_C, _H), jnp.float32),
            pltpu.VMEM((_C, _H), jnp.float32),
            pltpu.VMEM((_C,), jnp.int32),
            pltpu.VMEM((_C,), jnp.int32),
            pltpu.SemaphoreType.DMA,
            pltpu.SemaphoreType.DMA,
        ],
    )


def _edge_pass(h, ea, src, dst):
    return _edge_pass_kernel()(h, ea, src, dst)


def _mlp_body(h_ref, a0_ref, a1_ref, w1_ref, b1_ref, w2_ref, b2_ref,
              sc_ref, sb_ref, o_ref):
    z0 = h_ref[...] + a0_ref[...] + a1_ref[...]
    t = jnp.dot(z0, w1_ref[...], preferred_element_type=jnp.float32)
    t = jnp.maximum(t + b1_ref[...], 0.0)
    z = jnp.dot(t, w2_ref[...], preferred_element_type=jnp.float32) + b2_ref[...]
    o_ref[...] = jnp.maximum(z * sc_ref[...] + sb_ref[...], 0.0)


def _mlp(h, a0, a1, w1, b1, w2, b2, bn_scale, bn_bias, blk=1000):
    h2 = w1.shape[1]
    return pl.pallas_call(
        _mlp_body,
        grid=(_N // blk,),
        in_specs=[
            pl.BlockSpec((blk, _H), lambda i: (i, 0)),
            pl.BlockSpec((blk, _H), lambda i: (i, 0)),
            pl.BlockSpec((blk, _H), lambda i: (i, 0)),
            pl.BlockSpec((_H, h2), lambda i: (0, 0)),
            pl.BlockSpec((1, h2), lambda i: (0, 0)),
            pl.BlockSpec((h2, _H), lambda i: (0, 0)),
            pl.BlockSpec((1, _H), lambda i: (0, 0)),
            pl.BlockSpec((1, _H), lambda i: (0, 0)),
            pl.BlockSpec((1, _H), lambda i: (0, 0)),
        ],
        out_specs=pl.BlockSpec((blk, _H), lambda i: (i, 0)),
        out_shape=jax.ShapeDtypeStruct((_N, _H), jnp.float32),
    )(h, a0, a1, w1, b1.reshape(1, h2), w2, b2.reshape(1, _H),
      bn_scale.reshape(1, _H), bn_bias.reshape(1, _H))


def _pool_max_body(h_ref, seg_ref, gw_ref, gb_ref, o_ref):
    i = pl.program_id(0)
    g = jnp.sum(h_ref[...] * gw_ref[...], axis=1, keepdims=True) + gb_ref[0, 0]
    mask = seg_ref[...] == lax.broadcasted_iota(jnp.int32, (1, _G), 1)
    gm = jnp.max(jnp.where(mask, g, -1e30), axis=0, keepdims=True)

    @pl.when(i == 0)
    def _():
        o_ref[...] = jnp.full((1, _G), -1e30, jnp.float32)

    o_ref[...] = jnp.maximum(o_ref[...], gm)


def _pool_sum_body(h_ref, seg_ref, gmax_ref, gw_ref, gb_ref, pw_ref, pb_ref,
                   o_ref, num_acc, den_acc):
    i = pl.program_id(0)

    @pl.when(i == 0)
    def _():
        num_acc[...] = jnp.zeros_like(num_acc)
        den_acc[...] = jnp.zeros_like(den_acc)

    h = h_ref[...]
    g = jnp.sum(h * gw_ref[...], axis=1, keepdims=True) + gb_ref[0, 0]
    maskf = (seg_ref[...] == lax.broadcasted_iota(jnp.int32, (1, _G), 1)
             ).astype(jnp.float32)
    gsel = jnp.sum(maskf * gmax_ref[...], axis=1, keepdims=True)
    e = jnp.exp(g - gsel)
    ht = jnp.dot(h, pw_ref[...], preferred_element_type=jnp.float32) + pb_ref[...]
    dn = (((0,), (0,)), ((), ()))
    num_acc[...] += lax.dot_general(maskf, e * ht, dn,
                                    preferred_element_type=jnp.float32)
    den_acc[...] += lax.dot_general(maskf, jnp.broadcast_to(e, h.shape), dn,
                                    preferred_element_type=jnp.float32)

    @pl.when(i == pl.num_programs(0) - 1)
    def _():
        o_ref[...] = num_acc[...] / (den_acc[...] + 1e-16)


def _pool(h, seg, gate_w, gate_b, pool_w, pool_b, blk=1000):
    gw = gate_w.reshape(1, _H)
    gb = gate_b.reshape(1, 1)
    gmax = pl.pallas_call(
        _pool_max_body,
        grid=(_N // blk,),
        in_specs=[
            pl.BlockSpec((blk, _H), lambda i: (i, 0)),
            pl.BlockSpec((blk, 1), lambda i: (i, 0)),
            pl.BlockSpec((1, _H), lambda i: (0, 0)),
            pl.BlockSpec((1, 1), lambda i: (0, 0)),
        ],
        out_specs=pl.BlockSpec((1, _G), lambda i: (0, 0)),
        out_shape=jax.ShapeDtypeStruct((1, _G), jnp.float32),
    )(h, seg, gw, gb)
    return pl.pallas_call(
        _pool_sum_body,
        grid=(_N // blk,),
        in_specs=[
            pl.BlockSpec((blk, _H), lambda i: (i, 0)),
            pl.BlockSpec((blk, 1), lambda i: (i, 0)),
            pl.BlockSpec((1, _G), lambda i: (0, 0)),
            pl.BlockSpec((1, _H), lambda i: (0, 0)),
            pl.BlockSpec((1, 1), lambda i: (0, 0)),
            pl.BlockSpec((_H, _H), lambda i: (0, 0)),
            pl.BlockSpec((1, _H), lambda i: (0, 0)),
        ],
        out_specs=pl.BlockSpec((_G, _H), lambda i: (0, 0)),
        out_shape=jax.ShapeDtypeStruct((_G, _H), jnp.float32),
        scratch_shapes=[
            pltpu.VMEM((_G, _H), jnp.float32),
            pltpu.VMEM((_G, _H), jnp.float32),
        ],
    )(h, seg, gmax, gw, gb, pool_w, pool_b.reshape(1, _H))


def kernel(x, edge_index, edge_attr, batch, node_W, node_b, edge_W, edge_b,
           W1, b1, W2, b2, bn_g, bn_b, gate_W, gate_b, pool_W, pool_b):
    src = edge_index[0].astype(jnp.int32)
    dst = edge_index[1].astype(jnp.int32)
    seg = batch.astype(jnp.int32).reshape(_N, 1)
    bn_scale = bn_g * lax.rsqrt(jnp.float32(1.0 + 1e-5))

    h = _proj(x, node_W, node_b, blk=1000)
    ea = _proj(edge_attr, edge_W, edge_b, blk=4000)

    for l in range(_L):
        a0, a1 = _edge_pass(h, ea, src, dst)
        h = _mlp(h, a0, a1, W1[l], b1[l], W2[l], b2[l], bn_scale[l], bn_b[l])

    return _pool(h, seg, gate_W, gate_b, pool_W, pool_b)


# speedup vs baseline: 4.3254x; 1.3175x over previous
import functools

import jax
import jax.numpy as jnp
from jax import lax
from jax.experimental import pallas as pl
from jax.experimental.pallas import tpu as pltpu
from jax.experimental.pallas import tpu_sc as plsc

_N = 10000
_E = 320000
_H = 128
_HH = _H // 2
_G = 64
_L = 4

_NC = 2
_NS = 16
_EPT = _E // _NS
_C = 80
_NCH = _EPT // _C
_RPT0 = 640
_RPT1 = _N - (_NS - 1) * _RPT0


def _proj_body(x_ref, w_ref, b_ref, o_ref):
    x = x_ref[...]
    x = jnp.where(jnp.isnan(x), 0.0, x)
    y = jnp.dot(x, w_ref[...], preferred_element_type=jnp.float32)
    z = jnp.maximum(y + b_ref[...], 0.0)
    o_ref[0] = z[:, :_HH]
    o_ref[1] = z[:, _HH:]


def _proj(x, w, b, blk):
    m, k = x.shape
    return pl.pallas_call(
        _proj_body,
        grid=(m // blk,),
        in_specs=[
            pl.BlockSpec((blk, k), lambda i: (i, 0)),
            pl.BlockSpec((k, _H), lambda i: (0, 0)),
            pl.BlockSpec((1, _H), lambda i: (0, 0)),
        ],
        out_specs=pl.BlockSpec((2, blk, _HH), lambda i: (0, i, 0)),
        out_shape=jax.ShapeDtypeStruct((2, m, _HH), jnp.float32),
    )(x, w, b.reshape(1, _H))


def _sc_edge_body(h2_hbm, ea2_hbm, src3_hbm, dst3_hbm, out_lo, out_hi,
                  aggr_sh, srcv, dstv, ea0, ea1, g0, g1,
                  se0, se1, sg0, sg1):
    c = lax.axis_index("c")
    s = lax.axis_index("s")
    eab = (ea0, ea1)
    gvb = (g0, g1)
    seb = (se0, se1)
    sgb = (sg0, sg1)

    def _zrow(r, carry):
        for j in range(_HH // 16):
            g0[r, pl.ds(j * 16, 16)] = jnp.zeros((16,), jnp.float32)
        return carry

    lax.fori_loop(0, _C, _zrow, 0)
    r0 = pl.multiple_of(s * _RPT0, 8)

    @pl.when(s < _NS - 1)
    def _():
        for k in range(_RPT0 // _C):
            pltpu.sync_copy(g0, aggr_sh.at[pl.ds(r0 + k * _C, _C)])

    @pl.when(s == _NS - 1)
    def _():
        for k in range(_RPT1 // _C):
            pltpu.sync_copy(g0, aggr_sh.at[pl.ds(r0 + k * _C, _C)])

    plsc.subcore_barrier()

    base0 = c * _E + s * _EPT
    pltpu.sync_copy(src3_hbm.at[s], srcv)
    pltpu.sync_copy(dst3_hbm.at[s], dstv)

    roff = c * _N

    def _shift(r, carry):
        for j in range(_C // 16):
            sl = pl.ds(j * 16, 16)
            srcv[r, sl] = srcv[r, sl] + roff
        return carry

    lax.fori_loop(0, _NCH, _shift, 0)

    def _fetch(i, b):
        off = pl.multiple_of(base0 + i * _C, 8)
        pltpu.async_copy(ea2_hbm.at[pl.ds(off, _C)], eab[b], seb[b])
        pltpu.async_copy(h2_hbm.at[srcv.at[i]], gvb[b], sgb[b])

    def _process(i, b):
        eav = eab[b]
        gv = gvb[b]
        pltpu.make_async_copy(ea2_hbm.at[pl.ds(pl.multiple_of(base0, 8), _C)],
                              eav, seb[b]).wait()
        pltpu.make_async_copy(h2_hbm.at[srcv.at[i]], gv, sgb[b]).wait()

        def _row(r, c2):
            for j in range(_HH // 16):
                sl = pl.ds(j * 16, 16)
                eav[r, sl] = jnp.maximum(eav[r, sl] + gv[r, sl], 0.0)
            return c2

        lax.fori_loop(0, _C, _row, 0)
        pltpu.sync_copy(eav, aggr_sh.at[dstv.at[i]], add=True)

    _fetch(0, 0)
    _fetch(1, 1)

    def _pair(p, carry):
        for b in range(2):
            i = 2 * p + b
            _process(i, b)

            @pl.when(i + 2 < _NCH)
            def _():
                _fetch(i + 2, b)
        return carry

    lax.fori_loop(0, _NCH // 2, _pair, 0)
    plsc.subcore_barrier()

    @pl.when((c == 0) & (s < _NS - 1))
    def _():
        pltpu.sync_copy(aggr_sh.at[pl.ds(r0, _RPT0)], out_lo.at[pl.ds(r0, _RPT0)])

    @pl.when((c == 0) & (s == _NS - 1))
    def _():
        pltpu.sync_copy(aggr_sh.at[pl.ds(r0, _RPT1)], out_lo.at[pl.ds(r0, _RPT1)])

    @pl.when((c == 1) & (s < _NS - 1))
    def _():
        pltpu.sync_copy(aggr_sh.at[pl.ds(r0, _RPT0)], out_hi.at[pl.ds(r0, _RPT0)])

    @pl.when((c == 1) & (s == _NS - 1))
    def _():
        pltpu.sync_copy(aggr_sh.at[pl.ds(r0, _RPT1)], out_hi.at[pl.ds(r0, _RPT1)])


@functools.cache
def _edge_pass_kernel():
    return pl.kernel(
        _sc_edge_body,
        out_type=(jax.ShapeDtypeStruct((_N, _HH), jnp.float32),
                  jax.ShapeDtypeStruct((_N, _HH), jnp.float32)),
        mesh=plsc.VectorSubcoreMesh(core_axis_name="c", subcore_axis_name="s",
                                    num_cores=_NC, num_subcores=_NS),
        scratch_types=[
            pltpu.VMEM_SHARED((_N, _HH), jnp.float32),
            pltpu.VMEM((_NCH, _C), jnp.int32),
            pltpu.VMEM((_NCH, _C), jnp.int32),
            pltpu.VMEM((_C, _HH), jnp.float32),
            pltpu.VMEM((_C, _HH), jnp.float32),
            pltpu.VMEM((_C, _HH), jnp.float32),
            pltpu.VMEM((_C, _HH), jnp.float32),
            pltpu.SemaphoreType.DMA,
            pltpu.SemaphoreType.DMA,
            pltpu.SemaphoreType.DMA,
            pltpu.SemaphoreType.DMA,
        ],
        compiler_params=pltpu.CompilerParams(use_tc_tiling_on_sc=False),
    )


def _edge_pass(h2, ea2, src3, dst3):
    return _edge_pass_kernel()(h2.reshape(2 * _N, _HH), ea2, src3, dst3)


def _mlp_body(h_ref, alo_ref, ahi_ref, w1_ref, b1_ref, w2_ref, b2_ref,
              sc_ref, sb_ref, o_ref):
    z0 = jnp.concatenate([h_ref[0] + alo_ref[...], h_ref[1] + ahi_ref[...]],
                         axis=1)
    t = jnp.dot(z0, w1_ref[...], preferred_element_type=jnp.float32)
    t = jnp.maximum(t + b1_ref[...], 0.0)
    z = jnp.dot(t, w2_ref[...], preferred_element_type=jnp.float32) + b2_ref[...]
    z = jnp.maximum(z * sc_ref[...] + sb_ref[...], 0.0)
    o_ref[0] = z[:, :_HH]
    o_ref[1] = z[:, _HH:]


def _mlp(h2, alo, ahi, w1, b1, w2, b2, bn_scale, bn_bias, blk=1000):
    h2dim = w1.shape[1]
    return pl.pallas_call(
        _mlp_body,
        grid=(_N // blk,),
        in_specs=[
            pl.BlockSpec((2, blk, _HH), lambda i: (0, i, 0)),
            pl.BlockSpec((blk, _HH), lambda i: (i, 0)),
            pl.BlockSpec((blk, _HH), lambda i: (i, 0)),
            pl.BlockSpec((_H, h2dim), lambda i: (0, 0)),
            pl.BlockSpec((1, h2dim), lambda i: (0, 0)),
            pl.BlockSpec((h2dim, _H), lambda i: (0, 0)),
            pl.BlockSpec((1, _H), lambda i: (0, 0)),
            pl.BlockSpec((1, _H), lambda i: (0, 0)),
            pl.BlockSpec((1, _H), lambda i: (0, 0)),
        ],
        out_specs=pl.BlockSpec((2, blk, _HH), lambda i: (0, i, 0)),
        out_shape=jax.ShapeDtypeStruct((2, _N, _HH), jnp.float32),
    )(h2, alo, ahi, w1, b1.reshape(1, h2dim), w2, b2.reshape(1, _H),
      bn_scale.reshape(1, _H), bn_bias.reshape(1, _H))


def _pool_max_body(h_ref, seg_ref, gw_ref, gb_ref, o_ref):
    i = pl.program_id(0)
    h = jnp.concatenate([h_ref[0], h_ref[1]], axis=1)
    g = jnp.sum(h * gw_ref[...], axis=1, keepdims=True) + gb_ref[0, 0]
    mask = seg_ref[...] == lax.broadcasted_iota(jnp.int32, (1, _G), 1)
    gm = jnp.max(jnp.where(mask, g, -1e30), axis=0, keepdims=True)

    @pl.when(i == 0)
    def _():
        o_ref[...] = jnp.full((1, _G), -1e30, jnp.float32)

    o_ref[...] = jnp.maximum(o_ref[...], gm)


def _pool_sum_body(h_ref, seg_ref, gmax_ref, gw_ref, gb_ref, pw_ref, pb_ref,
                   o_ref, num_acc, den_acc):
    i = pl.program_id(0)

    @pl.when(i == 0)
    def _():
        num_acc[...] = jnp.zeros_like(num_acc)
        den_acc[...] = jnp.zeros_like(den_acc)

    h = jnp.concatenate([h_ref[0], h_ref[1]], axis=1)
    g = jnp.sum(h * gw_ref[...], axis=1, keepdims=True) + gb_ref[0, 0]
    maskf = (seg_ref[...] == lax.broadcasted_iota(jnp.int32, (1, _G), 1)
             ).astype(jnp.float32)
    gsel = jnp.sum(maskf * gmax_ref[...], axis=1, keepdims=True)
    e = jnp.exp(g - gsel)
    ht = jnp.dot(h, pw_ref[...], preferred_element_type=jnp.float32) + pb_ref[...]
    dn = (((0,), (0,)), ((), ()))
    num_acc[...] += lax.dot_general(maskf, e * ht, dn,
                                    preferred_element_type=jnp.float32)
    den_acc[...] += lax.dot_general(maskf, jnp.broadcast_to(e, h.shape), dn,
                                    preferred_element_type=jnp.float32)

    @pl.when(i == pl.num_programs(0) - 1)
    def _():
        o_ref[...] = num_acc[...] / (den_acc[...] + 1e-16)


def _pool(h2, seg, gate_w, gate_b, pool_w, pool_b, blk=1000):
    gw = gate_w.reshape(1, _H)
    gb = gate_b.reshape(1, 1)
    hspec = pl.BlockSpec((2, blk, _HH), lambda i: (0, i, 0))
    gmax = pl.pallas_call(
        _pool_max_body,
        grid=(_N // blk,),
        in_specs=[
            hspec,
            pl.BlockSpec((blk, 1), lambda i: (i, 0)),
            pl.BlockSpec((1, _H), lambda i: (0, 0)),
            pl.BlockSpec((1, 1), lambda i: (0, 0)),
        ],
        out_specs=pl.BlockSpec((1, _G), lambda i: (0, 0)),
        out_shape=jax.ShapeDtypeStruct((1, _G), jnp.float32),
    )(h2, seg, gw, gb)
    return pl.pallas_call(
        _pool_sum_body,
        grid=(_N // blk,),
        in_specs=[
            hspec,
            pl.BlockSpec((blk, 1), lambda i: (i, 0)),
            pl.BlockSpec((1, _G), lambda i: (0, 0)),
            pl.BlockSpec((1, _H), lambda i: (0, 0)),
            pl.BlockSpec((1, 1), lambda i: (0, 0)),
            pl.BlockSpec((_H, _H), lambda i: (0, 0)),
            pl.BlockSpec((1, _H), lambda i: (0, 0)),
        ],
        out_specs=pl.BlockSpec((_G, _H), lambda i: (0, 0)),
        out_shape=jax.ShapeDtypeStruct((_G, _H), jnp.float32),
        scratch_shapes=[
            pltpu.VMEM((_G, _H), jnp.float32),
            pltpu.VMEM((_G, _H), jnp.float32),
        ],
    )(h2, seg, gmax, gw, gb, pool_w, pool_b.reshape(1, _H))


def kernel(x, edge_index, edge_attr, batch, node_W, node_b, edge_W, edge_b,
           W1, b1, W2, b2, bn_g, bn_b, gate_W, gate_b, pool_W, pool_b):
    src = edge_index[0].astype(jnp.int32).reshape(_NS, _NCH, _C)
    dst = edge_index[1].astype(jnp.int32).reshape(_NS, _NCH, _C)
    seg = batch.astype(jnp.int32).reshape(_N, 1)
    bn_scale = bn_g * lax.rsqrt(jnp.float32(1.0 + 1e-5))

    h2 = _proj(x, node_W, node_b, blk=1000)
    ea2 = _proj(edge_attr, edge_W, edge_b, blk=4000)
    ea2f = ea2.reshape(2 * _E, _HH)

    for l in range(_L):
        alo, ahi = _edge_pass(h2, ea2f, src, dst)
        h2 = _mlp(h2, alo, ahi, W1[l], b1[l], W2[l], b2[l],
                  bn_scale[l], bn_b[l])

    return _pool(h2, seg, gate_W, gate_b, pool_W, pool_b)
